# baseline (speedup 1.0000x reference)
import dataclasses
import functools

import jax
import jax.numpy as jnp
from jax import lax
from jax.experimental import pallas as pl
from jax.experimental.pallas import tpu as pltpu
from jax.experimental.pallas import tpu_sc as plsc

N = 320000
D = 128
M = 10000

NC = 2
NS = 16
NW = NC * NS
CHUNK = 128
NCHUNKS = N // CHUNK
SEG_PER = 312
SLABR = 336
TBL = 40
NSLOT = 2
SUBCH = 1
BIGCH = SUBCH * CHUNK
NBLOCKS = N // BIGCH
NITER = (NBLOCKS + NSLOT - 1) // NSLOT

BLK = 2560


def _gate_weight_kernel(x_ref, idx_ref, w_ref, b_ref, y_ref, e_ref, o_ref):
    x = x_ref[...]
    g = jnp.sum(x * w_ref[...], axis=1, keepdims=True) + b_ref[0, 0]
    e = jnp.exp(g)
    y_ref[...] = x * e
    lane = lax.broadcasted_iota(jnp.int32, (x.shape[0], 16), 1)
    e_ref[...] = jnp.where(lane == 0, e, 0.0)

    b = pl.program_id(0)

    @pl.when(b == 0)
    def _():
        o_ref[...] = jnp.zeros_like(o_ref)

    iv = idx_ref[0]
    t = lax.broadcasted_iota(jnp.int32, (TBL, BLK), 0)
    thr = jnp.minimum(t * SEG_PER, M)
    mask = (jnp.broadcast_to(iv, (TBL, BLK)) < thr).astype(jnp.int32)
    cnt = jnp.sum(mask, axis=1, keepdims=True)
    o_ref[...] += jnp.broadcast_to(cnt, (TBL, 128))


def _combine_kernel(p_ref, s_ref, wm_ref, bm_ref, o_ref):
    p = p_ref[...]
    sden = s_ref[...][:, 0:1]
    acc = jnp.dot(p, wm_ref[...], precision=jax.lax.Precision.HIGHEST)
    o_ref[...] = (acc + sden * bm_ref[...]) / (sden + 1e-10)


def _make_sc_scatter():
    mesh = plsc.VectorSubcoreMesh(core_axis_name="c", subcore_axis_name="s")
    cp = pltpu.CompilerParams()
    if "needs_layout_passes" in pltpu.CompilerParams.__dataclass_fields__:
        cp = dataclasses.replace(cp, needs_layout_passes=False)

    @functools.partial(
        pl.kernel,
        mesh=mesh,
        compiler_params=cp,
        out_type=[
            jax.ShapeDtypeStruct((M, D), jnp.float32),
            jax.ShapeDtypeStruct((M, 16), jnp.float32),
        ],
        scratch_types=(
            [
                pltpu.VMEM_SHARED((NS * SLABR, D), jnp.float32),
                pltpu.VMEM_SHARED((NS * SLABR, 16), jnp.float32),
            ]
            + [pltpu.VMEM((BIGCH, D), jnp.float32)] * NSLOT
            + [pltpu.VMEM((BIGCH, 16), jnp.float32)] * NSLOT
            + [pltpu.VMEM((SUBCH, 1, CHUNK), jnp.int32)] * NSLOT
            + [pltpu.VMEM((CHUNK,), jnp.int32)] * (SUBCH * NSLOT)
            + [pltpu.VMEM((2, 1, 128), jnp.int32)]
            + [pltpu.SemaphoreType.DMA] * (2 * NSLOT)
        ),
    )
    def sc_scatter(y_hbm, e_hbm, idx_hbm, bnd_hbm, zy_hbm, ze_hbm,
                   py_hbm, pe_hbm, accy, acce, *scr):
        ybufs = scr[0:NSLOT]
        ebufs = scr[NSLOT:2 * NSLOT]
        idxrs = scr[2 * NSLOT:3 * NSLOT]
        nil = 3 * NSLOT
        ilocs = [scr[nil + j * SUBCH:nil + (j + 1) * SUBCH]
                 for j in range(NSLOT)]
        bsm = scr[nil + SUBCH * NSLOT]
        lsems = scr[nil + SUBCH * NSLOT + 1:nil + SUBCH * NSLOT + 1 + NSLOT]
        ssems = scr[nil + SUBCH * NSLOT + 1 + NSLOT:
                    nil + SUBCH * NSLOT + 1 + 2 * NSLOT]
        c = lax.axis_index("c")
        s = lax.axis_index("s")
        wid = s * NC + c
        slab = s * SLABR
        trash = slab + SLABR - 1

        pltpu.sync_copy(bnd_hbm.at[pl.ds(wid, 2)], bsm)
        lo = jnp.max(bsm[0, 0, pl.ds(0, 16)])
        hi = jnp.max(bsm[1, 0, pl.ds(0, 16)])
        lo_seg = wid * SEG_PER
        nseg = jnp.where(wid == NW - 1, M - (NW - 1) * SEG_PER, SEG_PER)
        hi_seg = lo_seg + nseg

        pltpu.sync_copy(zy_hbm, accy.at[pl.ds(slab, SLABR)])
        pltpu.sync_copy(ze_hbm, acce.at[pl.ds(slab, SLABR)])

        c0 = lax.div(lo, BIGCH)
        c1 = lax.div(hi + BIGCH - 1, BIGCH)

        def issue_loads(bi, ybuf, ebuf, idxr, sem):
            rs = pl.ds(bi * BIGCH, BIGCH)
            pltpu.async_copy(idx_hbm.at[pl.ds(bi * SUBCH, SUBCH)], idxr, sem)
            pltpu.async_copy(y_hbm.at[rs], ybuf, sem)
            pltpu.async_copy(e_hbm.at[rs], ebuf, sem)

        def wait_loads(bi, ybuf, ebuf, idxr, sem):
            rs = pl.ds(bi * BIGCH, BIGCH)
            pltpu.make_async_copy(idx_hbm.at[pl.ds(bi * SUBCH, SUBCH)],
                                  idxr, sem).wait()
            pltpu.make_async_copy(y_hbm.at[rs], ybuf, sem).wait()
            pltpu.make_async_copy(e_hbm.at[rs], ebuf, sem).wait()

        def remap_and_scatter(ybuf, ebuf, idxr, iloc, sem):
            for j in range(SUBCH):
                for g in range(CHUNK // 16):
                    v = idxr[j, 0, pl.ds(g * 16, 16)]
                    inr = jnp.logical_and(v >= lo_seg, v < hi_seg)
                    lv = jnp.where(inr, v - lo_seg + slab, trash)
                    iloc[j][pl.ds(g * 16, 16)] = lv

            for j in range(SUBCH):
                ss = pl.ds(j * CHUNK, CHUNK)
                pltpu.async_copy(ybuf.at[ss], accy.at[iloc[j]], sem, add=True)
                pltpu.async_copy(ebuf.at[ss], acce.at[iloc[j]], sem, add=True)

        def wait_scatter(ybuf, ebuf, iloc, sem):
            for j in range(SUBCH):
                ss = pl.ds(j * CHUNK, CHUNK)
                pltpu.make_async_copy(ybuf.at[ss], accy.at[iloc[j]],
                                      sem).wait()
                pltpu.make_async_copy(ebuf.at[ss], acce.at[iloc[j]],
                                      sem).wait()

        def on(bi):
            return jnp.logical_and(bi >= c0, bi < c1)

        @pl.loop(0, NITER)
        def _(t):
            base = NSLOT * t
            for k in range(NSLOT):
                ck = base + k

                @pl.when(on(ck - NSLOT))
                def _(k=k):
                    wait_scatter(ybufs[k], ebufs[k], ilocs[k], ssems[k])

                @pl.when(on(ck))
                def _(k=k, ck=ck):
                    issue_loads(ck, ybufs[k], ebufs[k], idxrs[k], lsems[k])

            for k in range(NSLOT):
                ck = base + k

                @pl.when(on(ck))
                def _(k=k, ck=ck):
                    wait_loads(ck, ybufs[k], ebufs[k], idxrs[k], lsems[k])
                    remap_and_scatter(ybufs[k], ebufs[k], idxrs[k], ilocs[k],
                                      ssems[k])

        for k in range(NSLOT):
            ck = (NITER - 1) * NSLOT + k

            @pl.when(on(ck))
            def _(k=k):
                wait_scatter(ybufs[k], ebufs[k], ilocs[k], ssems[k])

        pltpu.sync_copy(accy.at[pl.ds(slab, SEG_PER)],
                        py_hbm.at[pl.ds(wid * SEG_PER, SEG_PER)])
        pltpu.sync_copy(acce.at[pl.ds(slab, SEG_PER)],
                        pe_hbm.at[pl.ds(wid * SEG_PER, SEG_PER)])

        @pl.when(wid == NW - 1)
        def _():
            ex = M - NW * SEG_PER
            pltpu.sync_copy(accy.at[pl.ds(slab + SEG_PER, ex)],
                            py_hbm.at[pl.ds(NW * SEG_PER, ex)])
            pltpu.sync_copy(acce.at[pl.ds(slab + SEG_PER, ex)],
                            pe_hbm.at[pl.ds(NW * SEG_PER, ex)])

    return sc_scatter


_sc_scatter_cache = []


def _get_sc_scatter():
    if not _sc_scatter_cache:
        _sc_scatter_cache.append(_make_sc_scatter())
    return _sc_scatter_cache[0]


@jax.jit
def kernel(x, index, Wg, bg, Wm, bm):
    w_row = Wg.reshape(1, D)
    bg2 = bg.reshape(1, 1)
    bm2 = bm.reshape(1, D)

    idx3 = index.reshape(N // BLK, 1, BLK)
    y, e16, bounds = pl.pallas_call(
        _gate_weight_kernel,
        grid=(N // BLK,),
        in_specs=[
            pl.BlockSpec((BLK, D), lambda i: (i, 0)),
            pl.BlockSpec((1, 1, BLK), lambda i: (i, 0, 0)),
            pl.BlockSpec((1, D), lambda i: (0, 0)),
            pl.BlockSpec((1, 1), lambda i: (0, 0)),
        ],
        out_specs=[
            pl.BlockSpec((BLK, D), lambda i: (i, 0)),
            pl.BlockSpec((BLK, 16), lambda i: (i, 0)),
            pl.BlockSpec((TBL, 128), lambda i: (0, 0)),
        ],
        out_shape=[
            jax.ShapeDtypeStruct((N, D), jnp.float32),
            jax.ShapeDtypeStruct((N, 16), jnp.float32),
            jax.ShapeDtypeStruct((TBL, 128), jnp.int32),
        ],
    )(x, idx3, w_row, bg2)

    idx2 = index.reshape(NCHUNKS, 1, CHUNK)
    bnd3 = bounds.reshape(TBL, 1, 128)
    zy = jnp.zeros((SLABR, D), jnp.float32)
    ze = jnp.zeros((SLABR, 16), jnp.float32)
    py, pe = _get_sc_scatter()(y, e16, idx2, bnd3, zy, ze)

    out = pl.pallas_call(
        _combine_kernel,
        in_specs=[
            pl.BlockSpec((M, D), lambda: (0, 0)),
            pl.BlockSpec((M, 16), lambda: (0, 0)),
            pl.BlockSpec((D, D), lambda: (0, 0)),
            pl.BlockSpec((1, D), lambda: (0, 0)),
        ],
        out_specs=pl.BlockSpec((M, D), lambda: (0, 0)),
        out_shape=jax.ShapeDtypeStruct((M, D), jnp.float32),
    )(py, pe, Wm, bm2)
    return out
